# baseline (speedup 1.0000x reference)
import jax
import jax.numpy as jnp
from jax.experimental import pallas as pl
from jax.experimental.pallas import tpu as pltpu

_BM = 1024
_BN = 512
_DN = (((1,), (1,)), ((), ()))


def _qlin_kernel(x_ref, pk_ref, s_ref, zp_ref, o_ref, q_ref, xsum_ref):
    n = pl.program_id(0)
    m = pl.program_id(1)

    @pl.when(m == 0)
    def _():
        p = pk_ref[...].astype(jnp.int32)
        w32 = (p & 15) | ((p & 0xF0) << 12) | 0x43004300
        q_ref[...] = pltpu.bitcast(w32, jnp.bfloat16)

    @pl.when(n == 0)
    def _():
        xsum_ref[pl.ds(m * _BM, _BM), :] = jnp.sum(
            x_ref[...].astype(jnp.float32), axis=1, keepdims=True
        )

    acc = jnp.dot(x_ref[...], q_ref[...], preferred_element_type=jnp.float32)
    s = s_ref[...]
    zpb = zp_ref[...] + jnp.float32(128.0)
    o_ref[...] = acc * s - xsum_ref[pl.ds(m * _BM, _BM), :] * (s * zpb)


@jax.jit
def kernel(x, packed_weights, scales, zero_points):
    B, IN_F = x.shape
    OUT_F = packed_weights.shape[0]
    Kp = IN_F // 2

    xde = x.astype(jnp.bfloat16)
    pk = packed_weights.astype(jnp.uint8).T
    s2 = scales.reshape(1, OUT_F)
    zp2 = zero_points.reshape(1, OUT_F)

    grid = (pl.cdiv(OUT_F, _BN), B // _BM)
    return pl.pallas_call(
        _qlin_kernel,
        out_shape=jax.ShapeDtypeStruct((B, OUT_F), jnp.float32),
        grid=grid,
        in_specs=[
            pl.BlockSpec((_BM, IN_F), lambda n, m: (m, 0)),
            pl.BlockSpec((Kp, _BN), lambda n, m: (0, n)),
            pl.BlockSpec((1, _BN), lambda n, m: (0, n)),
            pl.BlockSpec((1, _BN), lambda n, m: (0, n)),
        ],
        out_specs=pl.BlockSpec((_BM, _BN), lambda n, m: (m, n)),
        scratch_shapes=[
            pltpu.VMEM((IN_F, _BN), jnp.bfloat16),
            pltpu.VMEM((B, 1), jnp.float32),
        ],
        compiler_params=pltpu.CompilerParams(
            dimension_semantics=("arbitrary", "arbitrary"),
        ),
        name="qlin_int4",
    )(xde, pk, s2, zp2)


# speedup vs baseline: 1.0636x; 1.0636x over previous
import jax
import jax.numpy as jnp
from jax.experimental import pallas as pl
from jax.experimental.pallas import tpu as pltpu

_BM = 1024
_BN = 1024
_DN = (((1,), (1,)), ((), ()))


def _qlin_kernel(x_ref, pk_ref, s_ref, zp_ref, o_ref, xsum_ref):
    n = pl.program_id(1)

    @pl.when(n == 0)
    def _():
        xsum_ref[...] = jnp.sum(
            x_ref[...].astype(jnp.float32), axis=1, keepdims=True
        )

    p = pk_ref[...].astype(jnp.int32)
    w32 = (p & 15) | ((p & 0xF0) << 12) | 0x43004300
    q = pltpu.bitcast(w32, jnp.bfloat16)
    acc = jnp.dot(x_ref[...], q, preferred_element_type=jnp.float32)
    s = s_ref[...]
    zpb = zp_ref[...] + jnp.float32(128.0)
    o_ref[...] = acc * s - xsum_ref[...] * (s * zpb)


@jax.jit
def kernel(x, packed_weights, scales, zero_points):
    B, IN_F = x.shape
    OUT_F = packed_weights.shape[0]
    Kp = IN_F // 2

    xde = x.astype(jnp.bfloat16)
    pk = packed_weights.astype(jnp.uint8).T
    s2 = scales.reshape(1, OUT_F)
    zp2 = zero_points.reshape(1, OUT_F)

    grid = (B // _BM, pl.cdiv(OUT_F, _BN))
    return pl.pallas_call(
        _qlin_kernel,
        out_shape=jax.ShapeDtypeStruct((B, OUT_F), jnp.float32),
        grid=grid,
        in_specs=[
            pl.BlockSpec((_BM, IN_F), lambda m, n: (m, 0)),
            pl.BlockSpec((Kp, _BN), lambda m, n: (0, n)),
            pl.BlockSpec((1, _BN), lambda m, n: (0, n)),
            pl.BlockSpec((1, _BN), lambda m, n: (0, n)),
        ],
        out_specs=pl.BlockSpec((_BM, _BN), lambda m, n: (m, n)),
        scratch_shapes=[pltpu.VMEM((_BM, 1), jnp.float32)],
        compiler_params=pltpu.CompilerParams(
            dimension_semantics=("arbitrary", "arbitrary"),
        ),
        name="qlin_int4",
    )(xde, pk, s2, zp2)


# speedup vs baseline: 1.3153x; 1.2366x over previous
import jax
import jax.numpy as jnp
from jax.experimental import pallas as pl
from jax.experimental.pallas import tpu as pltpu

_BM = 1024
_BN = 1024
_DN = (((1,), (1,)), ((), ()))


def _qlin_kernel(x_ref, pk_ref, s_ref, zp_ref, o_ref, xsum_ref):
    n = pl.program_id(1)

    @pl.when(n == 0)
    def _():
        xsum_ref[...] = jnp.sum(
            x_ref[...].astype(jnp.float32), axis=1, keepdims=True
        )

    p = pk_ref[...].T
    w32 = (p & 15) | ((p & 0xF0) << 12) | 0x43004300
    q = pltpu.bitcast(w32, jnp.bfloat16)
    acc = jnp.dot(x_ref[...], q, preferred_element_type=jnp.float32)
    s = s_ref[...]
    zpb = zp_ref[...] + jnp.float32(128.0)
    o_ref[...] = acc * s - xsum_ref[...] * (s * zpb)


@jax.jit
def kernel(x, packed_weights, scales, zero_points):
    B, IN_F = x.shape
    OUT_F = packed_weights.shape[0]
    Kp = IN_F // 2

    xde = x.astype(jnp.bfloat16)
    s2 = scales.reshape(1, OUT_F)
    zp2 = zero_points.reshape(1, OUT_F)

    grid = (B // _BM, pl.cdiv(OUT_F, _BN))
    return pl.pallas_call(
        _qlin_kernel,
        out_shape=jax.ShapeDtypeStruct((B, OUT_F), jnp.float32),
        grid=grid,
        in_specs=[
            pl.BlockSpec((_BM, IN_F), lambda m, n: (m, 0)),
            pl.BlockSpec((_BN, Kp), lambda m, n: (n, 0)),
            pl.BlockSpec((1, _BN), lambda m, n: (0, n)),
            pl.BlockSpec((1, _BN), lambda m, n: (0, n)),
        ],
        out_specs=pl.BlockSpec((_BM, _BN), lambda m, n: (m, n)),
        scratch_shapes=[pltpu.VMEM((_BM, 1), jnp.float32)],
        compiler_params=pltpu.CompilerParams(
            dimension_semantics=("arbitrary", "arbitrary"),
        ),
        name="qlin_int4",
    )(xde, packed_weights, s2, zp2)
